# baseline (speedup 1.0000x reference)
import functools

import jax
import jax.numpy as jnp
from jax import lax
from jax.experimental import pallas as pl
from jax.experimental.pallas import tpu as pltpu
from jax.experimental.pallas import tpu_sc as plsc

NUM_USERS = 100000
NUM_ITEMS = 100000
D = 32
B = 16384
HIST = 50
NG = 20
GEN = 64
DENSE = 8
PAD_IDX = NUM_ITEMS
USER_PAD_IDX = NUM_USERS
NROWS = NUM_USERS + 1


COLS_BLK = 8192


def _proj_body(utT, ugT, itT, mgT, gnT, gwT, w_ue, w_ug, w_ie, w_g, w_gen,
               w_pc, w_pd, ucomb, uprojd, icomb, iprojc):
    f32 = jnp.float32

    def dot(a, b):
        return lax.dot_general(a, b, (((1,), (0,)), ((), ())),
                               preferred_element_type=f32)

    gv_row = lax.dot_general(w_g[...], gwT[...], (((1,), (1,)), ((), ())),
                             preferred_element_type=f32)
    ucomb[...] = (dot(w_ue[...], utT[...]) + dot(w_ug[...], ugT[...]))[0]
    uprojd[...] = dot(w_pd[...], utT[...])[0]
    icomb[...] = (dot(w_ie[...], itT[...]) + dot(gv_row, mgT[...])
                  + dot(w_gen[...], gnT[...]))[0]
    iprojc[...] = dot(w_pc[...], itT[...])[0]


def _run_projections(user_table, user_genome, item_table, movie_genres,
                     genome, genre_W, w_ue, w_ug, w_ie, w_g, w_gen, w_pc,
                     w_pd):
    grid = (pl.cdiv(NROWS, COLS_BLK),)
    col_spec = lambda k: pl.BlockSpec((k, COLS_BLK), lambda i: (0, i))
    full_spec = lambda a, b: pl.BlockSpec((a, b), lambda i: (0, 0))
    out_spec = pl.BlockSpec((COLS_BLK,), lambda i: (i,))
    out_sd = jax.ShapeDtypeStruct((NROWS,), jnp.float32)
    return pl.pallas_call(
        _proj_body,
        grid=grid,
        in_specs=[
            col_spec(D), col_spec(GEN), col_spec(D), col_spec(NG),
            col_spec(GEN), full_spec(NG, D), full_spec(1, D),
            full_spec(1, GEN), full_spec(1, D), full_spec(1, D),
            full_spec(1, GEN), full_spec(1, D), full_spec(1, D),
        ],
        out_specs=[out_spec] * 4,
        out_shape=[out_sd] * 4,
    )(user_table.T, user_genome.T, item_table.T, movie_genres.T, genome.T,
      genre_W.T, w_ue, w_ug, w_ie, w_g, w_gen, w_pc, w_pd)


def _dense_body(dT, w_d, b_ref, out):
    dp = lax.dot_general(w_d[...], dT[...], (((1,), (0,)), ((), ())),
                         preferred_element_type=jnp.float32)
    out[...] = dp[0] + b_ref[0, 0]


def _run_dense_part(dense, w_dense, head_b):
    blk = 8192
    return pl.pallas_call(
        _dense_body,
        grid=(B // blk,),
        in_specs=[
            pl.BlockSpec((DENSE, blk), lambda i: (0, i)),
            pl.BlockSpec((1, DENSE), lambda i: (0, 0)),
            pl.BlockSpec((1, 1), lambda i: (0, 0)),
        ],
        out_specs=pl.BlockSpec((blk,), lambda i: (i,)),
        out_shape=jax.ShapeDtypeStruct((B,), jnp.float32),
    )(dense.T, w_dense, head_b.reshape(1, 1))


NC = 2
NS = 16
L = 16
NW = NC * NS
BPW = B // NW
CHUNK = 128
NCH = BPW // CHUNK


def _sc_body(uids_hbm, mids_hbm, densepart_hbm, uhist_hbm, uhrat_hbm,
             ihist_hbm, ihrat_hbm, ucomb_hbm, icomb_hbm, iprojc_hbm,
             uprojd_hbm, wmisc_hbm, out_hbm, uid_v, mid_v, fidx_v, histflat_v,
             ratflat_v, projflat_v, div_v, mod_v, ucomb_v, icomb_v,
             densepart_v, out_v, wmisc_v, sem1, sem2, sem3, sem4):
    wid = lax.axis_index("s") * NC + lax.axis_index("c")
    base = wid * BPW
    pltpu.sync_copy(uids_hbm.at[pl.ds(base, BPW)], uid_v)
    pltpu.sync_copy(mids_hbm.at[pl.ds(base, BPW)], mid_v)
    pltpu.sync_copy(densepart_hbm.at[pl.ds(base, BPW)], densepart_v)
    pltpu.sync_copy(wmisc_hbm, wmisc_v)
    cp_uc = pltpu.async_copy(ucomb_hbm.at[uid_v], ucomb_v, sem1)
    cp_ic = pltpu.async_copy(icomb_hbm.at[mid_v], icomb_v, sem2)

    iota = lax.iota(jnp.int32, L)
    zero = jnp.zeros((L,), jnp.float32)
    wm = wmisc_v[...]
    NFLAT = CHUNK * HIST // L

    def dm_body(g, _):
        val = jnp.full((L,), g * L, jnp.int32) + iota
        div_v[pl.ds(g * L, L)] = val // HIST
        mod_v[pl.ds(g * L, L)] = (val % HIST) * NROWS
        return _

    lax.fori_loop(0, NFLAT, dm_body, None)

    def do_side(idx_v, histflat_hbm, ratflat_hbm, proj_hbm, pad_val, w_rat,
                first):
        for ch in range(NCH):
            def fidx_body(g, _):
                sl = pl.ds(g * L, L)
                rows = div_v[sl] + (ch * CHUNK)
                uv = plsc.load_gather(idx_v, [rows])
                fidx_v[sl] = uv + mod_v[sl]
                return _

            lax.fori_loop(0, NFLAT, fidx_body, None)
            g1 = pltpu.async_copy(histflat_hbm.at[fidx_v], histflat_v, sem3)
            g2 = pltpu.async_copy(ratflat_hbm.at[fidx_v], ratflat_v, sem4)
            g1.wait()
            g2.wait()
            g3 = pltpu.async_copy(proj_hbm.at[histflat_v], projflat_v, sem3)
            g3.wait()
            for bg in range(CHUNK // L):
                base50 = (jnp.full((L,), bg * L, jnp.int32) + iota) * HIST

                def hbody(h, carry):
                    cnt, rsum, psum = carry
                    fidx = base50 + h
                    hv = plsc.load_gather(histflat_v, [fidx])
                    valid = hv != pad_val
                    cnt = cnt + jnp.where(valid, 1.0, 0.0)
                    rv = plsc.load_gather(ratflat_v, [fidx])
                    rsum = rsum + jnp.where(valid, rv, 0.0)
                    psum = psum + plsc.load_gather(projflat_v, [fidx])
                    return cnt, rsum, psum

                cnt, rsum, psum = lax.fori_loop(0, HIST, hbody,
                                                (zero, zero, zero))
                cnt = jnp.maximum(cnt, 1.0)
                contrib = (psum + w_rat * rsum) / cnt
                off = ch * CHUNK + bg * L
                if first:
                    out_v[pl.ds(off, L)] = contrib
                else:
                    out_v[pl.ds(off, L)] = out_v[pl.ds(off, L)] + contrib

    do_side(uid_v, uhist_hbm, uhrat_hbm, iprojc_hbm, PAD_IDX, wm[0], True)
    do_side(mid_v, ihist_hbm, ihrat_hbm, uprojd_hbm, USER_PAD_IDX, wm[1],
            False)

    cp_uc.wait()
    cp_ic.wait()
    for bg in range(BPW // L):
        sl = pl.ds(bg * L, L)
        out_v[sl] = (out_v[sl] + ucomb_v[sl] + icomb_v[sl] + densepart_v[sl])
    pltpu.sync_copy(out_v, out_hbm.at[pl.ds(base, BPW)])


def _sc_run(uids, mids, densepart, uhist, uhrat, ihist, ihrat, ucomb, icomb,
            iprojc, uprojd, wmisc):
    mesh = plsc.VectorSubcoreMesh(core_axis_name="c", subcore_axis_name="s",
                                  num_cores=NC, num_subcores=NS)
    f = pl.kernel(
        _sc_body,
        out_type=jax.ShapeDtypeStruct((B,), jnp.float32),
        mesh=mesh,
        compiler_params=pltpu.CompilerParams(
            needs_layout_passes=False,
            use_tc_tiling_on_sc=False,
        ),
        scratch_types=[
            pltpu.VMEM((BPW,), jnp.int32),
            pltpu.VMEM((BPW,), jnp.int32),
            pltpu.VMEM((CHUNK * HIST,), jnp.int32),
            pltpu.VMEM((CHUNK * HIST,), jnp.int32),
            pltpu.VMEM((CHUNK * HIST,), jnp.float32),
            pltpu.VMEM((CHUNK * HIST,), jnp.float32),
            pltpu.VMEM((CHUNK * HIST,), jnp.int32),
            pltpu.VMEM((CHUNK * HIST,), jnp.int32),
            pltpu.VMEM((BPW,), jnp.float32),
            pltpu.VMEM((BPW,), jnp.float32),
            pltpu.VMEM((BPW,), jnp.float32),
            pltpu.VMEM((BPW,), jnp.float32),
            pltpu.VMEM((L,), jnp.float32),
            pltpu.SemaphoreType.DMA,
            pltpu.SemaphoreType.DMA,
            pltpu.SemaphoreType.DMA,
            pltpu.SemaphoreType.DMA,
        ],
    )
    return f(uids, mids, densepart, uhist, uhrat, ihist, ihrat, ucomb, icomb,
             iprojc, uprojd, wmisc)


def kernel(uids, mids, dense, user_table, item_table, genre_W, head_W,
           head_b, user_hist, user_hist_rat, item_hist, item_hist_rat,
           movie_genres, genome, user_genome):
    i32 = jnp.int32
    uids = uids.astype(i32)
    mids = mids.astype(i32)
    user_hist = user_hist.astype(i32)
    item_hist = item_hist.astype(i32)

    w = head_W[0]
    row = lambda a, b: w[a:b].reshape(1, -1)
    w_ue = row(0, 32)
    w_ie = row(32, 64)
    w_pc = row(64, 96)
    w_u_rat = w[96]
    w_pd = row(97, 129)
    w_i_rat = w[129]
    w_g = row(130, 162)
    w_dense = row(162, 170)
    w_gen = row(170, 234)
    w_ug = row(234, 298)

    ucomb, uprojd, icomb, iprojc = _run_projections(
        user_table, user_genome, item_table, movie_genres, genome, genre_W,
        w_ue, w_ug, w_ie, w_g, w_gen, w_pc, w_pd)
    densepart = _run_dense_part(dense, w_dense, head_b)

    wmisc = jnp.concatenate([
        jnp.stack([w_u_rat, w_i_rat]),
        jnp.zeros((14,), jnp.float32),
    ])

    return _sc_run(uids, mids, densepart, user_hist.T.reshape(-1),
                   user_hist_rat.T.reshape(-1), item_hist.T.reshape(-1),
                   item_hist_rat.T.reshape(-1), ucomb, icomb, iprojc, uprojd,
                   wmisc)


# speedup vs baseline: 3.5249x; 3.5249x over previous
import functools

import jax
import jax.numpy as jnp
from jax import lax
from jax.experimental import pallas as pl
from jax.experimental.pallas import tpu as pltpu
from jax.experimental.pallas import tpu_sc as plsc

NUM_USERS = 100000
NUM_ITEMS = 100000
D = 32
B = 16384
HIST = 50
NG = 20
GEN = 64
DENSE = 8
PAD_IDX = NUM_ITEMS
USER_PAD_IDX = NUM_USERS
NROWS = NUM_USERS + 1


COLS_BLK = 8192


def _proj_body(utT, ugT, itT, mgT, gnT, gwT, w_ue, w_ug, w_ie, w_g, w_gen,
               w_pc, w_pd, ucomb, uprojd, icomb, iprojc):
    f32 = jnp.float32

    def dot(a, b):
        return lax.dot_general(a, b, (((1,), (0,)), ((), ())),
                               preferred_element_type=f32)

    gv_row = lax.dot_general(w_g[...], gwT[...], (((1,), (1,)), ((), ())),
                             preferred_element_type=f32)
    ucomb[...] = (dot(w_ue[...], utT[...]) + dot(w_ug[...], ugT[...]))[0]
    uprojd[...] = dot(w_pd[...], utT[...])[0]
    icomb[...] = (dot(w_ie[...], itT[...]) + dot(gv_row, mgT[...])
                  + dot(w_gen[...], gnT[...]))[0]
    iprojc[...] = dot(w_pc[...], itT[...])[0]


def _run_projections(user_table, user_genome, item_table, movie_genres,
                     genome, genre_W, w_ue, w_ug, w_ie, w_g, w_gen, w_pc,
                     w_pd):
    grid = (pl.cdiv(NROWS, COLS_BLK),)
    col_spec = lambda k: pl.BlockSpec((k, COLS_BLK), lambda i: (0, i))
    full_spec = lambda a, b: pl.BlockSpec((a, b), lambda i: (0, 0))
    out_spec = pl.BlockSpec((COLS_BLK,), lambda i: (i,))
    out_sd = jax.ShapeDtypeStruct((NROWS,), jnp.float32)
    return pl.pallas_call(
        _proj_body,
        grid=grid,
        in_specs=[
            col_spec(D), col_spec(GEN), col_spec(D), col_spec(NG),
            col_spec(GEN), full_spec(NG, D), full_spec(1, D),
            full_spec(1, GEN), full_spec(1, D), full_spec(1, D),
            full_spec(1, GEN), full_spec(1, D), full_spec(1, D),
        ],
        out_specs=[out_spec] * 4,
        out_shape=[out_sd] * 4,
    )(user_table.T, user_genome.T, item_table.T, movie_genres.T, genome.T,
      genre_W.T, w_ue, w_ug, w_ie, w_g, w_gen, w_pc, w_pd)


NP = 100096
HR = 8
HIST_PAD = 56


def _untile_body(aT, bT, a_o, b_o):
    a = aT[...]
    b = bT[...]
    for r in range(HR):
        a_o[pl.ds(r * NP, NP)] = a[r]
        b_o[pl.ds(r * NP, NP)] = b[r]


def _untile_pair(hist_t, rat_t):
    in_spec = pl.BlockSpec((HR, NP), lambda h: (h, 0))
    out_spec = pl.BlockSpec((HR * NP,), lambda h: (h,))
    sd = lambda dt: jax.ShapeDtypeStruct((HIST_PAD * NP,), dt)
    return pl.pallas_call(
        _untile_body,
        grid=(HIST_PAD // HR,),
        in_specs=[in_spec] * 2,
        out_specs=[out_spec] * 2,
        out_shape=[sd(jnp.int32), sd(jnp.float32)],
    )(hist_t.T, rat_t.T)


def _run_untile(uhist, uhrat, ihist, ihrat):
    uh_o, ur_o = _untile_pair(uhist, uhrat)
    ih_o, ir_o = _untile_pair(ihist, ihrat)
    return uh_o, ur_o, ih_o, ir_o


def _dense_body(dT, w_d, b_ref, out):
    dp = lax.dot_general(w_d[...], dT[...], (((1,), (0,)), ((), ())),
                         preferred_element_type=jnp.float32)
    out[...] = dp[0] + b_ref[0, 0]


def _run_dense_part(dense, w_dense, head_b):
    blk = 8192
    return pl.pallas_call(
        _dense_body,
        grid=(B // blk,),
        in_specs=[
            pl.BlockSpec((DENSE, blk), lambda i: (0, i)),
            pl.BlockSpec((1, DENSE), lambda i: (0, 0)),
            pl.BlockSpec((1, 1), lambda i: (0, 0)),
        ],
        out_specs=pl.BlockSpec((blk,), lambda i: (i,)),
        out_shape=jax.ShapeDtypeStruct((B,), jnp.float32),
    )(dense.T, w_dense, head_b.reshape(1, 1))


NC = 2
NS = 16
L = 16
NW = NC * NS
BPW = B // NW
CHUNK = 128
NCH = BPW // CHUNK


def _sc_body(uids_hbm, mids_hbm, densepart_hbm, uhist_hbm, uhrat_hbm,
             ihist_hbm, ihrat_hbm, ucomb_hbm, icomb_hbm, iprojc_hbm,
             uprojd_hbm, wmisc_hbm, out_hbm, uid_v, mid_v, fidx_v, histflat_v,
             ratflat_v, projflat_v, div_v, mod_v, ucomb_v, icomb_v,
             densepart_v, out_v, wmisc_v, sem1, sem2, sem3, sem4):
    wid = lax.axis_index("s") * NC + lax.axis_index("c")
    base = wid * BPW
    pltpu.sync_copy(uids_hbm.at[pl.ds(base, BPW)], uid_v)
    pltpu.sync_copy(mids_hbm.at[pl.ds(base, BPW)], mid_v)
    pltpu.sync_copy(densepart_hbm.at[pl.ds(base, BPW)], densepart_v)
    pltpu.sync_copy(wmisc_hbm, wmisc_v)
    cp_uc = pltpu.async_copy(ucomb_hbm.at[uid_v], ucomb_v, sem1)
    cp_ic = pltpu.async_copy(icomb_hbm.at[mid_v], icomb_v, sem2)

    iota = lax.iota(jnp.int32, L)
    zero = jnp.zeros((L,), jnp.float32)
    wm = wmisc_v[...]
    NFLAT = CHUNK * HIST // L

    def dm_body(g, _):
        val = jnp.full((L,), g * L, jnp.int32) + iota
        div_v[pl.ds(g * L, L)] = val // HIST
        mod_v[pl.ds(g * L, L)] = (val % HIST) * NP
        return _

    lax.fori_loop(0, NFLAT, dm_body, None)

    def do_side(idx_v, histflat_hbm, ratflat_hbm, proj_hbm, pad_val, w_rat,
                first):
        for ch in range(NCH):
            def fidx_body(g, _):
                sl = pl.ds(g * L, L)
                rows = div_v[sl] + (ch * CHUNK)
                uv = plsc.load_gather(idx_v, [rows])
                fidx_v[sl] = uv + mod_v[sl]
                return _

            lax.fori_loop(0, NFLAT, fidx_body, None)
            g1 = pltpu.async_copy(histflat_hbm.at[fidx_v], histflat_v, sem3)
            g2 = pltpu.async_copy(ratflat_hbm.at[fidx_v], ratflat_v, sem4)
            g1.wait()
            g2.wait()
            g3 = pltpu.async_copy(proj_hbm.at[histflat_v], projflat_v, sem3)
            g3.wait()
            for bg in range(CHUNK // L):
                base50 = (jnp.full((L,), bg * L, jnp.int32) + iota) * HIST

                def hbody(h, carry):
                    cnt, rsum, psum = carry
                    fidx = base50 + h
                    hv = plsc.load_gather(histflat_v, [fidx])
                    valid = hv != pad_val
                    cnt = cnt + jnp.where(valid, 1.0, 0.0)
                    rv = plsc.load_gather(ratflat_v, [fidx])
                    rsum = rsum + jnp.where(valid, rv, 0.0)
                    psum = psum + plsc.load_gather(projflat_v, [fidx])
                    return cnt, rsum, psum

                cnt, rsum, psum = lax.fori_loop(0, HIST, hbody,
                                                (zero, zero, zero))
                cnt = jnp.maximum(cnt, 1.0)
                contrib = (psum + w_rat * rsum) / cnt
                off = ch * CHUNK + bg * L
                if first:
                    out_v[pl.ds(off, L)] = contrib
                else:
                    out_v[pl.ds(off, L)] = out_v[pl.ds(off, L)] + contrib

    do_side(uid_v, uhist_hbm, uhrat_hbm, iprojc_hbm, PAD_IDX, wm[0], True)
    do_side(mid_v, ihist_hbm, ihrat_hbm, uprojd_hbm, USER_PAD_IDX, wm[1],
            False)

    cp_uc.wait()
    cp_ic.wait()
    for bg in range(BPW // L):
        sl = pl.ds(bg * L, L)
        out_v[sl] = (out_v[sl] + ucomb_v[sl] + icomb_v[sl] + densepart_v[sl])
    pltpu.sync_copy(out_v, out_hbm.at[pl.ds(base, BPW)])


def _sc_run(uids, mids, densepart, uhist, uhrat, ihist, ihrat, ucomb, icomb,
            iprojc, uprojd, wmisc):
    mesh = plsc.VectorSubcoreMesh(core_axis_name="c", subcore_axis_name="s",
                                  num_cores=NC, num_subcores=NS)
    f = pl.kernel(
        _sc_body,
        out_type=jax.ShapeDtypeStruct((B,), jnp.float32),
        mesh=mesh,
        compiler_params=pltpu.CompilerParams(
            needs_layout_passes=False,
            use_tc_tiling_on_sc=False,
        ),
        scratch_types=[
            pltpu.VMEM((BPW,), jnp.int32),
            pltpu.VMEM((BPW,), jnp.int32),
            pltpu.VMEM((CHUNK * HIST,), jnp.int32),
            pltpu.VMEM((CHUNK * HIST,), jnp.int32),
            pltpu.VMEM((CHUNK * HIST,), jnp.float32),
            pltpu.VMEM((CHUNK * HIST,), jnp.float32),
            pltpu.VMEM((CHUNK * HIST,), jnp.int32),
            pltpu.VMEM((CHUNK * HIST,), jnp.int32),
            pltpu.VMEM((BPW,), jnp.float32),
            pltpu.VMEM((BPW,), jnp.float32),
            pltpu.VMEM((BPW,), jnp.float32),
            pltpu.VMEM((BPW,), jnp.float32),
            pltpu.VMEM((L,), jnp.float32),
            pltpu.SemaphoreType.DMA,
            pltpu.SemaphoreType.DMA,
            pltpu.SemaphoreType.DMA,
            pltpu.SemaphoreType.DMA,
        ],
    )
    return f(uids, mids, densepart, uhist, uhrat, ihist, ihrat, ucomb, icomb,
             iprojc, uprojd, wmisc)


def kernel(uids, mids, dense, user_table, item_table, genre_W, head_W,
           head_b, user_hist, user_hist_rat, item_hist, item_hist_rat,
           movie_genres, genome, user_genome):
    i32 = jnp.int32
    uids = uids.astype(i32)
    mids = mids.astype(i32)
    user_hist = user_hist.astype(i32)
    item_hist = item_hist.astype(i32)

    w = head_W[0]
    row = lambda a, b: w[a:b].reshape(1, -1)
    w_ue = row(0, 32)
    w_ie = row(32, 64)
    w_pc = row(64, 96)
    w_u_rat = w[96]
    w_pd = row(97, 129)
    w_i_rat = w[129]
    w_g = row(130, 162)
    w_dense = row(162, 170)
    w_gen = row(170, 234)
    w_ug = row(234, 298)

    ucomb, uprojd, icomb, iprojc = _run_projections(
        user_table, user_genome, item_table, movie_genres, genome, genre_W,
        w_ue, w_ug, w_ie, w_g, w_gen, w_pc, w_pd)
    densepart = _run_dense_part(dense, w_dense, head_b)

    wmisc = jnp.concatenate([
        jnp.stack([w_u_rat, w_i_rat]),
        jnp.zeros((14,), jnp.float32),
    ])

    uh_f, ur_f, ih_f, ir_f = _run_untile(user_hist, user_hist_rat, item_hist,
                                         item_hist_rat)
    return _sc_run(uids, mids, densepart, uh_f, ur_f, ih_f, ir_f, ucomb,
                   icomb, iprojc, uprojd, wmisc)


# speedup vs baseline: 5.0453x; 1.4313x over previous
import functools

import jax
import jax.numpy as jnp
from jax import lax
from jax.experimental import pallas as pl
from jax.experimental.pallas import tpu as pltpu
from jax.experimental.pallas import tpu_sc as plsc

NUM_USERS = 100000
NUM_ITEMS = 100000
D = 32
B = 16384
HIST = 50
NG = 20
GEN = 64
DENSE = 8
PAD_IDX = NUM_ITEMS
USER_PAD_IDX = NUM_USERS
NROWS = NUM_USERS + 1


COLS_BLK = 8192


def _proj_body(utT, ugT, itT, mgT, gnT, gwT, w_ue, w_ug, w_ie, w_g, w_gen,
               w_pc, w_pd, wrats, ucnt, urs, icnt, irs, ucomb, uprojd, icomb,
               iprojc, uinv, iinv):
    f32 = jnp.float32

    def dot(a, b):
        return lax.dot_general(a, b, (((1,), (0,)), ((), ())),
                               preferred_element_type=f32)

    gv_row = lax.dot_general(w_g[...], gwT[...], (((1,), (1,)), ((), ())),
                             preferred_element_type=f32)
    ucntc = jnp.maximum(ucnt[...], 1.0)
    icntc = jnp.maximum(icnt[...], 1.0)
    uinv[...] = 1.0 / ucntc
    iinv[...] = 1.0 / icntc
    w_u_rat = wrats[0, 0]
    w_i_rat = wrats[0, 1]
    ucomb[...] = ((dot(w_ue[...], utT[...]) + dot(w_ug[...], ugT[...]))[0]
                  + w_u_rat * urs[...] / ucntc)
    uprojd[...] = dot(w_pd[...], utT[...])[0]
    icomb[...] = ((dot(w_ie[...], itT[...]) + dot(gv_row, mgT[...])
                   + dot(w_gen[...], gnT[...]))[0]
                  + w_i_rat * irs[...] / icntc)
    iprojc[...] = dot(w_pc[...], itT[...])[0]


def _run_projections(user_table, user_genome, item_table, movie_genres,
                     genome, genre_W, w_ue, w_ug, w_ie, w_g, w_gen, w_pc,
                     w_pd, wrats, ucnt, urs, icnt, irs):
    grid = (pl.cdiv(NROWS, COLS_BLK),)
    col_spec = lambda k: pl.BlockSpec((k, COLS_BLK), lambda i: (0, i))
    full_spec = lambda a, b: pl.BlockSpec((a, b), lambda i: (0, 0))
    vec_spec = pl.BlockSpec((COLS_BLK,), lambda i: (i,))
    out_sd = jax.ShapeDtypeStruct((NROWS,), jnp.float32)
    return pl.pallas_call(
        _proj_body,
        grid=grid,
        in_specs=[
            col_spec(D), col_spec(GEN), col_spec(D), col_spec(NG),
            col_spec(GEN), full_spec(NG, D), full_spec(1, D),
            full_spec(1, GEN), full_spec(1, D), full_spec(1, D),
            full_spec(1, GEN), full_spec(1, D), full_spec(1, D),
            full_spec(1, 2), vec_spec, vec_spec, vec_spec, vec_spec,
        ],
        out_specs=[vec_spec] * 6,
        out_shape=[out_sd] * 6,
    )(user_table.T, user_genome.T, item_table.T, movie_genres.T, genome.T,
      genre_W.T, w_ue, w_ug, w_ie, w_g, w_gen, w_pc, w_pd, wrats, ucnt, urs,
      icnt, irs)


NP = 100096
HR = 8
HIST_PAD = 56


def _untile_body(aT, bT, a_o, cnt_o, rs_o):
    g = pl.program_id(0)
    a = aT[...]
    b = bT[...]
    for r in range(HR):
        a_o[pl.ds(r * NP, NP)] = a[r]
    row_ok = (lax.broadcasted_iota(jnp.int32, (HR, NP), 0) + g * HR) < HIST
    valid = (a != PAD_IDX) & row_ok
    vcnt = jnp.sum(valid.astype(jnp.float32), axis=0)
    vrs = jnp.sum(jnp.where(valid, b, 0.0), axis=0)

    @pl.when(g == 0)
    def _init():
        cnt_o[...] = vcnt
        rs_o[...] = vrs

    @pl.when(g != 0)
    def _acc():
        cnt_o[...] = cnt_o[...] + vcnt
        rs_o[...] = rs_o[...] + vrs


def _untile_pair(hist_t, rat_t):
    in_spec = pl.BlockSpec((HR, NP), lambda h: (h, 0))
    return pl.pallas_call(
        _untile_body,
        grid=(HIST_PAD // HR,),
        in_specs=[in_spec] * 2,
        out_specs=[
            pl.BlockSpec((HR * NP,), lambda h: (h,)),
            pl.BlockSpec((NP,), lambda h: (0,)),
            pl.BlockSpec((NP,), lambda h: (0,)),
        ],
        out_shape=[
            jax.ShapeDtypeStruct((HIST_PAD * NP,), jnp.int32),
            jax.ShapeDtypeStruct((NP,), jnp.float32),
            jax.ShapeDtypeStruct((NP,), jnp.float32),
        ],
    )(hist_t.T, rat_t.T)


def _dense_body(dT, w_d, b_ref, out):
    dp = lax.dot_general(w_d[...], dT[...], (((1,), (0,)), ((), ())),
                         preferred_element_type=jnp.float32)
    out[...] = dp[0] + b_ref[0, 0]


def _run_dense_part(dense, w_dense, head_b):
    blk = 8192
    return pl.pallas_call(
        _dense_body,
        grid=(B // blk,),
        in_specs=[
            pl.BlockSpec((DENSE, blk), lambda i: (0, i)),
            pl.BlockSpec((1, DENSE), lambda i: (0, 0)),
            pl.BlockSpec((1, 1), lambda i: (0, 0)),
        ],
        out_specs=pl.BlockSpec((blk,), lambda i: (i,)),
        out_shape=jax.ShapeDtypeStruct((B,), jnp.float32),
    )(dense.T, w_dense, head_b.reshape(1, 1))


NC = 2
NS = 16
L = 16
NW = NC * NS
BPW = B // NW
CHUNK = 128
NCH = BPW // CHUNK


def _sc_body(uids_hbm, mids_hbm, densepart_hbm, uhf_hbm, ihf_hbm, ucomb_hbm,
             icomb_hbm, uinv_hbm, iinv_hbm, iprojc_hbm, uprojd_hbm, out_hbm,
             uid_v, mid_v, fidx_v, histflat_v, div_v, mod_v, ucomb_v,
             icomb_v, uinv_v, iinv_v, densepart_v, out_v, proj_v, sem1, sem2,
             sem3, sem4):
    wid = lax.axis_index("s") * NC + lax.axis_index("c")
    base = wid * BPW
    pltpu.sync_copy(uids_hbm.at[pl.ds(base, BPW)], uid_v)
    pltpu.sync_copy(mids_hbm.at[pl.ds(base, BPW)], mid_v)
    pltpu.sync_copy(densepart_hbm.at[pl.ds(base, BPW)], densepart_v)
    cp_uc = pltpu.async_copy(ucomb_hbm.at[uid_v], ucomb_v, sem1)
    cp_ic = pltpu.async_copy(icomb_hbm.at[mid_v], icomb_v, sem2)
    cp_ui = pltpu.async_copy(uinv_hbm.at[uid_v], uinv_v, sem3)
    cp_ii = pltpu.async_copy(iinv_hbm.at[mid_v], iinv_v, sem4)

    iota = lax.iota(jnp.int32, L)
    zero = jnp.zeros((L,), jnp.float32)
    NFLAT = CHUNK * HIST // L

    def dm_body(g, _):
        val = jnp.full((L,), g * L, jnp.int32) + iota
        div_v[pl.ds(g * L, L)] = val // HIST
        mod_v[pl.ds(g * L, L)] = (val % HIST) * NP
        return _

    lax.fori_loop(0, NFLAT, dm_body, None)
    cp_ui.wait()
    cp_ii.wait()

    def do_side(idx_v, hf_hbm, proj_hbm, inv_v, first):
        pltpu.sync_copy(proj_hbm, proj_v)
        for ch in range(NCH):
            def fidx_body(g, _):
                sl = pl.ds(g * L, L)
                rows = div_v[sl] + (ch * CHUNK)
                uv = plsc.load_gather(idx_v, [rows])
                fidx_v[sl] = uv + mod_v[sl]
                return _

            lax.fori_loop(0, NFLAT, fidx_body, None)
            g1 = pltpu.async_copy(hf_hbm.at[fidx_v], histflat_v, sem3)
            g1.wait()
            for bg in range(CHUNK // L):
                base50 = (jnp.full((L,), bg * L, jnp.int32) + iota) * HIST

                def hbody(h, psum):
                    hv = plsc.load_gather(histflat_v, [base50 + h])
                    return psum + plsc.load_gather(proj_v, [hv])

                psum = lax.fori_loop(0, HIST, hbody, zero)
                off = ch * CHUNK + bg * L
                contrib = psum * inv_v[pl.ds(off, L)]
                if first:
                    out_v[pl.ds(off, L)] = contrib
                else:
                    out_v[pl.ds(off, L)] = out_v[pl.ds(off, L)] + contrib

    do_side(uid_v, uhf_hbm, iprojc_hbm, uinv_v, True)
    do_side(mid_v, ihf_hbm, uprojd_hbm, iinv_v, False)

    cp_uc.wait()
    cp_ic.wait()
    for bg in range(BPW // L):
        sl = pl.ds(bg * L, L)
        out_v[sl] = (out_v[sl] + ucomb_v[sl] + icomb_v[sl] + densepart_v[sl])
    pltpu.sync_copy(out_v, out_hbm.at[pl.ds(base, BPW)])


def _sc_run(uids, mids, densepart, uhf, ihf, ucomb, icomb, uinv, iinv,
            iprojc, uprojd):
    mesh = plsc.VectorSubcoreMesh(core_axis_name="c", subcore_axis_name="s",
                                  num_cores=NC, num_subcores=NS)
    f = pl.kernel(
        _sc_body,
        out_type=jax.ShapeDtypeStruct((B,), jnp.float32),
        mesh=mesh,
        compiler_params=pltpu.CompilerParams(
            needs_layout_passes=False,
            use_tc_tiling_on_sc=False,
        ),
        scratch_types=[
            pltpu.VMEM((BPW,), jnp.int32),
            pltpu.VMEM((BPW,), jnp.int32),
            pltpu.VMEM((CHUNK * HIST,), jnp.int32),
            pltpu.VMEM((CHUNK * HIST,), jnp.int32),
            pltpu.VMEM((CHUNK * HIST,), jnp.int32),
            pltpu.VMEM((CHUNK * HIST,), jnp.int32),
            pltpu.VMEM((BPW,), jnp.float32),
            pltpu.VMEM((BPW,), jnp.float32),
            pltpu.VMEM((BPW,), jnp.float32),
            pltpu.VMEM((BPW,), jnp.float32),
            pltpu.VMEM((BPW,), jnp.float32),
            pltpu.VMEM((BPW,), jnp.float32),
            pltpu.VMEM((NROWS,), jnp.float32),
            pltpu.SemaphoreType.DMA,
            pltpu.SemaphoreType.DMA,
            pltpu.SemaphoreType.DMA,
            pltpu.SemaphoreType.DMA,
        ],
    )
    return f(uids, mids, densepart, uhf, ihf, ucomb, icomb, uinv, iinv,
             iprojc, uprojd)


def kernel(uids, mids, dense, user_table, item_table, genre_W, head_W,
           head_b, user_hist, user_hist_rat, item_hist, item_hist_rat,
           movie_genres, genome, user_genome):
    i32 = jnp.int32
    uids = uids.astype(i32)
    mids = mids.astype(i32)
    user_hist = user_hist.astype(i32)
    item_hist = item_hist.astype(i32)

    w = head_W[0]
    row = lambda a, b: w[a:b].reshape(1, -1)
    w_ue = row(0, 32)
    w_ie = row(32, 64)
    w_pc = row(64, 96)
    w_u_rat = w[96]
    w_pd = row(97, 129)
    w_i_rat = w[129]
    w_g = row(130, 162)
    w_dense = row(162, 170)
    w_gen = row(170, 234)
    w_ug = row(234, 298)

    uh_f, ucnt, urs = _untile_pair(user_hist, user_hist_rat)
    ih_f, icnt, irs = _untile_pair(item_hist, item_hist_rat)

    wrats = jnp.stack([w_u_rat, w_i_rat]).reshape(1, 2)
    ucomb, uprojd, icomb, iprojc, uinv, iinv = _run_projections(
        user_table, user_genome, item_table, movie_genres, genome, genre_W,
        w_ue, w_ug, w_ie, w_g, w_gen, w_pc, w_pd, wrats, ucnt, urs, icnt,
        irs)
    densepart = _run_dense_part(dense, w_dense, head_b)

    return _sc_run(uids, mids, densepart, uh_f, ih_f, ucomb, icomb, uinv,
                   iinv, iprojc, uprojd)


# speedup vs baseline: 5.7748x; 1.1446x over previous
import functools

import jax
import jax.numpy as jnp
from jax import lax
from jax.experimental import pallas as pl
from jax.experimental.pallas import tpu as pltpu
from jax.experimental.pallas import tpu_sc as plsc

NUM_USERS = 100000
NUM_ITEMS = 100000
D = 32
B = 16384
HIST = 50
NG = 20
GEN = 64
DENSE = 8
PAD_IDX = NUM_ITEMS
USER_PAD_IDX = NUM_USERS
NROWS = NUM_USERS + 1


COLS_BLK = 8192


def _proj_body(utT, ugT, itT, mgT, gnT, gwT, w_ue, w_ug, w_ie, w_g, w_gen,
               w_pc, w_pd, wrats, ucnt, urs, icnt, irs, ucomb, uprojd, icomb,
               iprojc, uinv, iinv):
    f32 = jnp.float32

    def dot(a, b):
        return lax.dot_general(a, b, (((1,), (0,)), ((), ())),
                               preferred_element_type=f32)

    gv_row = lax.dot_general(w_g[...], gwT[...], (((1,), (1,)), ((), ())),
                             preferred_element_type=f32)
    ucntc = jnp.maximum(ucnt[...], 1.0)
    icntc = jnp.maximum(icnt[...], 1.0)
    uinv[...] = 1.0 / ucntc
    iinv[...] = 1.0 / icntc
    w_u_rat = wrats[0, 0]
    w_i_rat = wrats[0, 1]
    ucomb[...] = ((dot(w_ue[...], utT[...]) + dot(w_ug[...], ugT[...]))[0]
                  + w_u_rat * urs[...] / ucntc)
    uprojd[...] = dot(w_pd[...], utT[...])[0]
    icomb[...] = ((dot(w_ie[...], itT[...]) + dot(gv_row, mgT[...])
                   + dot(w_gen[...], gnT[...]))[0]
                  + w_i_rat * irs[...] / icntc)
    iprojc[...] = dot(w_pc[...], itT[...])[0]


def _run_projections(user_table, user_genome, item_table, movie_genres,
                     genome, genre_W, w_ue, w_ug, w_ie, w_g, w_gen, w_pc,
                     w_pd, wrats, ucnt, urs, icnt, irs):
    grid = (pl.cdiv(NROWS, COLS_BLK),)
    col_spec = lambda k: pl.BlockSpec((k, COLS_BLK), lambda i: (0, i))
    full_spec = lambda a, b: pl.BlockSpec((a, b), lambda i: (0, 0))
    vec_spec = pl.BlockSpec((COLS_BLK,), lambda i: (i,))
    out_sd = jax.ShapeDtypeStruct((NROWS,), jnp.float32)
    return pl.pallas_call(
        _proj_body,
        grid=grid,
        in_specs=[
            col_spec(D), col_spec(GEN), col_spec(D), col_spec(NG),
            col_spec(GEN), full_spec(NG, D), full_spec(1, D),
            full_spec(1, GEN), full_spec(1, D), full_spec(1, D),
            full_spec(1, GEN), full_spec(1, D), full_spec(1, D),
            full_spec(1, 2), vec_spec, vec_spec, vec_spec, vec_spec,
        ],
        out_specs=[vec_spec] * 6,
        out_shape=[out_sd] * 6,
    )(user_table.T, user_genome.T, item_table.T, movie_genres.T, genome.T,
      genre_W.T, w_ue, w_ug, w_ie, w_g, w_gen, w_pc, w_pd, wrats, ucnt, urs,
      icnt, irs)


NP = 100096
HR = 8
HIST_PAD = 56


def _untile_body(aT, bT, a_o, cnt_o, rs_o):
    g = pl.program_id(0)
    a = aT[...]
    b = bT[...]
    for r in range(HR):
        a_o[pl.ds(r * NP, NP)] = a[r]
    row_ok = (lax.broadcasted_iota(jnp.int32, (HR, NP), 0) + g * HR) < HIST
    valid = (a != PAD_IDX) & row_ok
    vcnt = jnp.sum(valid.astype(jnp.float32), axis=0)
    vrs = jnp.sum(jnp.where(valid, b, 0.0), axis=0)

    @pl.when(g == 0)
    def _init():
        cnt_o[...] = vcnt
        rs_o[...] = vrs

    @pl.when(g != 0)
    def _acc():
        cnt_o[...] = cnt_o[...] + vcnt
        rs_o[...] = rs_o[...] + vrs


def _untile_pair(hist_t, rat_t):
    in_spec = pl.BlockSpec((HR, NP), lambda h: (h, 0))
    return pl.pallas_call(
        _untile_body,
        grid=(HIST_PAD // HR,),
        in_specs=[in_spec] * 2,
        out_specs=[
            pl.BlockSpec((HR * NP,), lambda h: (h,)),
            pl.BlockSpec((NP,), lambda h: (0,)),
            pl.BlockSpec((NP,), lambda h: (0,)),
        ],
        out_shape=[
            jax.ShapeDtypeStruct((HIST_PAD * NP,), jnp.int32),
            jax.ShapeDtypeStruct((NP,), jnp.float32),
            jax.ShapeDtypeStruct((NP,), jnp.float32),
        ],
    )(hist_t.T, rat_t.T)


def _dense_body(dT, w_d, b_ref, out):
    dp = lax.dot_general(w_d[...], dT[...], (((1,), (0,)), ((), ())),
                         preferred_element_type=jnp.float32)
    out[...] = dp[0] + b_ref[0, 0]


def _run_dense_part(dense, w_dense, head_b):
    blk = 8192
    return pl.pallas_call(
        _dense_body,
        grid=(B // blk,),
        in_specs=[
            pl.BlockSpec((DENSE, blk), lambda i: (0, i)),
            pl.BlockSpec((1, DENSE), lambda i: (0, 0)),
            pl.BlockSpec((1, 1), lambda i: (0, 0)),
        ],
        out_specs=pl.BlockSpec((blk,), lambda i: (i,)),
        out_shape=jax.ShapeDtypeStruct((B,), jnp.float32),
    )(dense.T, w_dense, head_b.reshape(1, 1))


NC = 2
NS = 16
L = 16
NW = NC * NS
BPW = B // NW
CHUNK = 128
NCH = BPW // CHUNK


def _sc_body(uids_hbm, mids_hbm, densepart_hbm, uhf_hbm, ihf_hbm, ucomb_hbm,
             icomb_hbm, uinv_hbm, iinv_hbm, iprojc_hbm, uprojd_hbm, out_hbm,
             uid_v, mid_v, fidx_v0, fidx_v1, histflat_v0, histflat_v1,
             ucomb_v, icomb_v, uinv_v, iinv_v, densepart_v, out_v, proj_v,
             sem1, sem2, sem3, sem4):
    wid = lax.axis_index("s") * NC + lax.axis_index("c")
    base = wid * BPW
    pltpu.sync_copy(uids_hbm.at[pl.ds(base, BPW)], uid_v)
    pltpu.sync_copy(mids_hbm.at[pl.ds(base, BPW)], mid_v)
    pltpu.sync_copy(densepart_hbm.at[pl.ds(base, BPW)], densepart_v)
    cp_uc = pltpu.async_copy(ucomb_hbm.at[uid_v], ucomb_v, sem1)
    cp_ic = pltpu.async_copy(icomb_hbm.at[mid_v], icomb_v, sem2)
    cp_ui = pltpu.async_copy(uinv_hbm.at[uid_v], uinv_v, sem3)
    cp_ii = pltpu.async_copy(iinv_hbm.at[mid_v], iinv_v, sem4)

    zero = jnp.zeros((L,), jnp.float32)
    fidx_b = [fidx_v0, fidx_v1]
    hist_b = [histflat_v0, histflat_v1]
    sems = [sem3, sem4]
    cp_ui.wait()
    cp_ii.wait()

    def build_fidx(idx_v, ch, buf):
        def fb(h, _):
            for g in range(CHUNK // L):
                sl = pl.ds(h * CHUNK + g * L, L)
                uv = idx_v[pl.ds(ch * CHUNK + g * L, L)]
                buf[sl] = uv + h * NP
            return _

        lax.fori_loop(0, HIST, fb, None)

    def do_side(idx_v, hf_hbm, proj_hbm, inv_v, first):
        pltpu.sync_copy(proj_hbm, proj_v)
        build_fidx(idx_v, 0, fidx_b[0])
        cps = [None, None]
        cps[0] = pltpu.async_copy(hf_hbm.at[fidx_b[0]], hist_b[0], sems[0])
        for ch in range(NCH):
            pb = ch % 2
            if ch + 1 < NCH:
                nb = (ch + 1) % 2
                build_fidx(idx_v, ch + 1, fidx_b[nb])
                cps[nb] = pltpu.async_copy(hf_hbm.at[fidx_b[nb]],
                                           hist_b[nb], sems[nb])
            cps[pb].wait()
            hbuf = hist_b[pb]
            for bg in range(CHUNK // L):

                def hbody(k, psum):
                    for m in range(5):
                        hv = hbuf[pl.ds((k * 5 + m) * CHUNK + bg * L, L)]
                        psum = psum + plsc.load_gather(proj_v, [hv])
                    return psum

                psum = lax.fori_loop(0, HIST // 5, hbody, zero)
                off = ch * CHUNK + bg * L
                contrib = psum * inv_v[pl.ds(off, L)]
                if first:
                    out_v[pl.ds(off, L)] = contrib
                else:
                    out_v[pl.ds(off, L)] = out_v[pl.ds(off, L)] + contrib

    do_side(uid_v, uhf_hbm, iprojc_hbm, uinv_v, True)
    do_side(mid_v, ihf_hbm, uprojd_hbm, iinv_v, False)

    cp_uc.wait()
    cp_ic.wait()
    for bg in range(BPW // L):
        sl = pl.ds(bg * L, L)
        out_v[sl] = (out_v[sl] + ucomb_v[sl] + icomb_v[sl] + densepart_v[sl])
    pltpu.sync_copy(out_v, out_hbm.at[pl.ds(base, BPW)])


def _sc_run(uids, mids, densepart, uhf, ihf, ucomb, icomb, uinv, iinv,
            iprojc, uprojd):
    mesh = plsc.VectorSubcoreMesh(core_axis_name="c", subcore_axis_name="s",
                                  num_cores=NC, num_subcores=NS)
    f = pl.kernel(
        _sc_body,
        out_type=jax.ShapeDtypeStruct((B,), jnp.float32),
        mesh=mesh,
        compiler_params=pltpu.CompilerParams(
            needs_layout_passes=False,
            use_tc_tiling_on_sc=False,
        ),
        scratch_types=[
            pltpu.VMEM((BPW,), jnp.int32),
            pltpu.VMEM((BPW,), jnp.int32),
            pltpu.VMEM((CHUNK * HIST,), jnp.int32),
            pltpu.VMEM((CHUNK * HIST,), jnp.int32),
            pltpu.VMEM((CHUNK * HIST,), jnp.int32),
            pltpu.VMEM((CHUNK * HIST,), jnp.int32),
            pltpu.VMEM((BPW,), jnp.float32),
            pltpu.VMEM((BPW,), jnp.float32),
            pltpu.VMEM((BPW,), jnp.float32),
            pltpu.VMEM((BPW,), jnp.float32),
            pltpu.VMEM((BPW,), jnp.float32),
            pltpu.VMEM((BPW,), jnp.float32),
            pltpu.VMEM((NROWS,), jnp.float32),
            pltpu.SemaphoreType.DMA,
            pltpu.SemaphoreType.DMA,
            pltpu.SemaphoreType.DMA,
            pltpu.SemaphoreType.DMA,
        ],
    )
    return f(uids, mids, densepart, uhf, ihf, ucomb, icomb, uinv, iinv,
             iprojc, uprojd)


def kernel(uids, mids, dense, user_table, item_table, genre_W, head_W,
           head_b, user_hist, user_hist_rat, item_hist, item_hist_rat,
           movie_genres, genome, user_genome):
    i32 = jnp.int32
    uids = uids.astype(i32)
    mids = mids.astype(i32)
    user_hist = user_hist.astype(i32)
    item_hist = item_hist.astype(i32)

    w = head_W[0]
    row = lambda a, b: w[a:b].reshape(1, -1)
    w_ue = row(0, 32)
    w_ie = row(32, 64)
    w_pc = row(64, 96)
    w_u_rat = w[96]
    w_pd = row(97, 129)
    w_i_rat = w[129]
    w_g = row(130, 162)
    w_dense = row(162, 170)
    w_gen = row(170, 234)
    w_ug = row(234, 298)

    uh_f, ucnt, urs = _untile_pair(user_hist, user_hist_rat)
    ih_f, icnt, irs = _untile_pair(item_hist, item_hist_rat)

    wrats = jnp.stack([w_u_rat, w_i_rat]).reshape(1, 2)
    ucomb, uprojd, icomb, iprojc, uinv, iinv = _run_projections(
        user_table, user_genome, item_table, movie_genres, genome, genre_W,
        w_ue, w_ug, w_ie, w_g, w_gen, w_pc, w_pd, wrats, ucnt, urs, icnt,
        irs)
    densepart = _run_dense_part(dense, w_dense, head_b)

    return _sc_run(uids, mids, densepart, uh_f, ih_f, ucomb, icomb, uinv,
                   iinv, iprojc, uprojd)


# speedup vs baseline: 5.9175x; 1.0247x over previous
import functools

import jax
import jax.numpy as jnp
from jax import lax
from jax.experimental import pallas as pl
from jax.experimental.pallas import tpu as pltpu
from jax.experimental.pallas import tpu_sc as plsc

NUM_USERS = 100000
NUM_ITEMS = 100000
D = 32
B = 16384
HIST = 50
NG = 20
GEN = 64
DENSE = 8
PAD_IDX = NUM_ITEMS
USER_PAD_IDX = NUM_USERS
NROWS = NUM_USERS + 1


COLS_BLK = 8192


_COL_SPEC = lambda k: pl.BlockSpec((k, COLS_BLK), lambda i: (0, i))
_FULL_SPEC = lambda a, b: pl.BlockSpec((a, b), lambda i: (0, 0))
_VEC_SPEC = pl.BlockSpec((COLS_BLK,), lambda i: (i,))
_VEC_SD = jax.ShapeDtypeStruct((NROWS,), jnp.float32)


def _dotrow(a, b):
    return lax.dot_general(a, b, (((1,), (0,)), ((), ())),
                           preferred_element_type=jnp.float32)


def _proj_u_body(utT, ugT, itT, w_ue, w_ug, w_pc, wrats, ucnt, urs, ucomb,
                 uinv, iprojc):
    ucntc = jnp.maximum(ucnt[...], 1.0)
    uinv[...] = 1.0 / ucntc
    ucomb[...] = ((_dotrow(w_ue[...], utT[...])
                   + _dotrow(w_ug[...], ugT[...]))[0]
                  + wrats[0, 0] * urs[...] / ucntc)
    iprojc[...] = _dotrow(w_pc[...], itT[...])[0]


def _run_proj_u(user_table, user_genome, item_table, w_ue, w_ug, w_pc, wrats,
                ucnt, urs):
    return pl.pallas_call(
        _proj_u_body,
        grid=(pl.cdiv(NROWS, COLS_BLK),),
        in_specs=[
            _COL_SPEC(D), _COL_SPEC(GEN), _COL_SPEC(D), _FULL_SPEC(1, D),
            _FULL_SPEC(1, GEN), _FULL_SPEC(1, D), _FULL_SPEC(1, 2),
            _VEC_SPEC, _VEC_SPEC,
        ],
        out_specs=[_VEC_SPEC] * 3,
        out_shape=[_VEC_SD] * 3,
    )(user_table.T, user_genome.T, item_table.T, w_ue, w_ug, w_pc, wrats,
      ucnt, urs)


def _proj_i_body(utT, itT, mgT, gnT, gwT, w_ie, w_g, w_gen, w_pd, wrats,
                 icnt, irs, icomb, iinv, uprojd):
    gv_row = lax.dot_general(w_g[...], gwT[...], (((1,), (1,)), ((), ())),
                             preferred_element_type=jnp.float32)
    icntc = jnp.maximum(icnt[...], 1.0)
    iinv[...] = 1.0 / icntc
    icomb[...] = ((_dotrow(w_ie[...], itT[...]) + _dotrow(gv_row, mgT[...])
                   + _dotrow(w_gen[...], gnT[...]))[0]
                  + wrats[0, 1] * irs[...] / icntc)
    uprojd[...] = _dotrow(w_pd[...], utT[...])[0]


def _run_proj_i(user_table, item_table, movie_genres, genome, genre_W, w_ie,
                w_g, w_gen, w_pd, wrats, icnt, irs):
    return pl.pallas_call(
        _proj_i_body,
        grid=(pl.cdiv(NROWS, COLS_BLK),),
        in_specs=[
            _COL_SPEC(D), _COL_SPEC(D), _COL_SPEC(NG), _COL_SPEC(GEN),
            _FULL_SPEC(NG, D), _FULL_SPEC(1, D), _FULL_SPEC(1, D),
            _FULL_SPEC(1, GEN), _FULL_SPEC(1, D), _FULL_SPEC(1, 2),
            _VEC_SPEC, _VEC_SPEC,
        ],
        out_specs=[_VEC_SPEC] * 3,
        out_shape=[_VEC_SD] * 3,
    )(user_table.T, item_table.T, movie_genres.T, genome.T, genre_W.T, w_ie,
      w_g, w_gen, w_pd, wrats, icnt, irs)


NP = 100096
HR = 8
HIST_PAD = 56


def _untile_body(aT, bT, a_o, cnt_o, rs_o):
    g = pl.program_id(0)
    a = aT[...]
    b = bT[...]
    for r in range(HR):
        a_o[pl.ds(r * NP, NP)] = a[r]
    row_ok = (lax.broadcasted_iota(jnp.int32, (HR, NP), 0) + g * HR) < HIST
    valid = (a != PAD_IDX) & row_ok
    vcnt = jnp.sum(valid.astype(jnp.float32), axis=0)
    vrs = jnp.sum(jnp.where(valid, b, 0.0), axis=0)

    @pl.when(g == 0)
    def _init():
        cnt_o[...] = vcnt
        rs_o[...] = vrs

    @pl.when(g != 0)
    def _acc():
        cnt_o[...] = cnt_o[...] + vcnt
        rs_o[...] = rs_o[...] + vrs


def _untile_pair(hist_t, rat_t):
    in_spec = pl.BlockSpec((HR, NP), lambda h: (h, 0))
    return pl.pallas_call(
        _untile_body,
        grid=(HIST_PAD // HR,),
        in_specs=[in_spec] * 2,
        out_specs=[
            pl.BlockSpec((HR * NP,), lambda h: (h,)),
            pl.BlockSpec((NP,), lambda h: (0,)),
            pl.BlockSpec((NP,), lambda h: (0,)),
        ],
        out_shape=[
            jax.ShapeDtypeStruct((HIST_PAD * NP,), jnp.int32),
            jax.ShapeDtypeStruct((NP,), jnp.float32),
            jax.ShapeDtypeStruct((NP,), jnp.float32),
        ],
    )(hist_t.T, rat_t.T)


def _dense_body(dT, w_d, b_ref, out):
    dp = lax.dot_general(w_d[...], dT[...], (((1,), (0,)), ((), ())),
                         preferred_element_type=jnp.float32)
    out[...] = dp[0] + b_ref[0, 0]


def _run_dense_part(dense, w_dense, head_b):
    blk = 8192
    return pl.pallas_call(
        _dense_body,
        grid=(B // blk,),
        in_specs=[
            pl.BlockSpec((DENSE, blk), lambda i: (0, i)),
            pl.BlockSpec((1, DENSE), lambda i: (0, 0)),
            pl.BlockSpec((1, 1), lambda i: (0, 0)),
        ],
        out_specs=pl.BlockSpec((blk,), lambda i: (i,)),
        out_shape=jax.ShapeDtypeStruct((B,), jnp.float32),
    )(dense.T, w_dense, head_b.reshape(1, 1))


NC = 2
NS = 16
L = 16
NW = NC * NS
BPW = B // NW
CHUNK = 128
NCH = BPW // CHUNK


def _sc_pool(idx_v, hf_hbm, inv_v, extra_v, out_v, proj_v, fidx_b, hist_b,
             sems):
    zero = jnp.zeros((L,), jnp.float32)

    def build_fidx(ch, buf):
        def fb(h, _):
            for g in range(CHUNK // L):
                sl = pl.ds(h * CHUNK + g * L, L)
                uv = idx_v[pl.ds(ch * CHUNK + g * L, L)]
                buf[sl] = uv + h * NP
            return _

        lax.fori_loop(0, HIST, fb, None)

    build_fidx(0, fidx_b[0])
    cps = [None, None]
    cps[0] = pltpu.async_copy(hf_hbm.at[fidx_b[0]], hist_b[0], sems[0])
    for ch in range(NCH):
        pb = ch % 2
        if ch + 1 < NCH:
            nb = (ch + 1) % 2
            build_fidx(ch + 1, fidx_b[nb])
            cps[nb] = pltpu.async_copy(hf_hbm.at[fidx_b[nb]], hist_b[nb],
                                       sems[nb])
        cps[pb].wait()
        hbuf = hist_b[pb]
        for bg in range(CHUNK // L):

            def hbody(k, psum):
                for m in range(5):
                    hv = hbuf[pl.ds((k * 5 + m) * CHUNK + bg * L, L)]
                    psum = psum + plsc.load_gather(proj_v, [hv])
                return psum

            psum = lax.fori_loop(0, HIST // 5, hbody, zero)
            off = ch * CHUNK + bg * L
            sl = pl.ds(off, L)
            out_v[sl] = psum * inv_v[sl] + extra_v[sl]


def _sc_u_body(uids_hbm, densepart_hbm, uhf_hbm, ucomb_hbm, uinv_hbm,
               iprojc_hbm, out_hbm, uid_v, fidx_v0, fidx_v1, histflat_v0,
               histflat_v1, ucomb_v, uinv_v, densepart_v, out_v, proj_v,
               sem1, sem3, sem4):
    wid = lax.axis_index("s") * NC + lax.axis_index("c")
    base = wid * BPW
    pltpu.sync_copy(uids_hbm.at[pl.ds(base, BPW)], uid_v)
    pltpu.sync_copy(densepart_hbm.at[pl.ds(base, BPW)], densepart_v)
    cp_uc = pltpu.async_copy(ucomb_hbm.at[uid_v], ucomb_v, sem1)
    cp_ui = pltpu.async_copy(uinv_hbm.at[uid_v], uinv_v, sem3)
    pltpu.sync_copy(iprojc_hbm, proj_v)
    cp_uc.wait()
    cp_ui.wait()
    for bg in range(BPW // L):
        sl = pl.ds(bg * L, L)
        densepart_v[sl] = densepart_v[sl] + ucomb_v[sl]
    _sc_pool(uid_v, uhf_hbm, uinv_v, densepart_v, out_v, proj_v,
             [fidx_v0, fidx_v1], [histflat_v0, histflat_v1], [sem3, sem4])
    pltpu.sync_copy(out_v, out_hbm.at[pl.ds(base, BPW)])


def _sc_i_body(mids_hbm, part_hbm, ihf_hbm, icomb_hbm, iinv_hbm, uprojd_hbm,
               out_hbm, mid_v, fidx_v0, fidx_v1, histflat_v0, histflat_v1,
               icomb_v, iinv_v, part_v, out_v, proj_v, sem1, sem3, sem4):
    wid = lax.axis_index("s") * NC + lax.axis_index("c")
    base = wid * BPW
    pltpu.sync_copy(mids_hbm.at[pl.ds(base, BPW)], mid_v)
    pltpu.sync_copy(part_hbm.at[pl.ds(base, BPW)], part_v)
    cp_ic = pltpu.async_copy(icomb_hbm.at[mid_v], icomb_v, sem1)
    cp_ii = pltpu.async_copy(iinv_hbm.at[mid_v], iinv_v, sem3)
    pltpu.sync_copy(uprojd_hbm, proj_v)
    cp_ic.wait()
    cp_ii.wait()
    for bg in range(BPW // L):
        sl = pl.ds(bg * L, L)
        part_v[sl] = part_v[sl] + icomb_v[sl]
    _sc_pool(mid_v, ihf_hbm, iinv_v, part_v, out_v, proj_v,
             [fidx_v0, fidx_v1], [histflat_v0, histflat_v1], [sem3, sem4])
    pltpu.sync_copy(out_v, out_hbm.at[pl.ds(base, BPW)])


_SC_SCRATCH = [
    pltpu.VMEM((BPW,), jnp.int32),
    pltpu.VMEM((CHUNK * HIST,), jnp.int32),
    pltpu.VMEM((CHUNK * HIST,), jnp.int32),
    pltpu.VMEM((CHUNK * HIST,), jnp.int32),
    pltpu.VMEM((CHUNK * HIST,), jnp.int32),
    pltpu.VMEM((BPW,), jnp.float32),
    pltpu.VMEM((BPW,), jnp.float32),
    pltpu.VMEM((BPW,), jnp.float32),
    pltpu.VMEM((BPW,), jnp.float32),
    pltpu.VMEM((NROWS,), jnp.float32),
    pltpu.SemaphoreType.DMA,
    pltpu.SemaphoreType.DMA,
    pltpu.SemaphoreType.DMA,
]


def _sc_run(uids, mids, densepart, uhf, ihf, ucomb, icomb, uinv, iinv,
            iprojc, uprojd):
    mesh = plsc.VectorSubcoreMesh(core_axis_name="c", subcore_axis_name="s",
                                  num_cores=NC, num_subcores=NS)
    params = pltpu.CompilerParams(
        needs_layout_passes=False,
        use_tc_tiling_on_sc=False,
    )
    out_t = jax.ShapeDtypeStruct((B,), jnp.float32)
    f_u = pl.kernel(_sc_u_body, out_type=out_t, mesh=mesh,
                    compiler_params=params, scratch_types=_SC_SCRATCH)
    part = f_u(uids, densepart, uhf, ucomb, uinv, iprojc)
    f_i = pl.kernel(_sc_i_body, out_type=out_t, mesh=mesh,
                    compiler_params=params, scratch_types=_SC_SCRATCH)
    return f_i(mids, part, ihf, icomb, iinv, uprojd)


def kernel(uids, mids, dense, user_table, item_table, genre_W, head_W,
           head_b, user_hist, user_hist_rat, item_hist, item_hist_rat,
           movie_genres, genome, user_genome):
    i32 = jnp.int32
    uids = uids.astype(i32)
    mids = mids.astype(i32)
    user_hist = user_hist.astype(i32)
    item_hist = item_hist.astype(i32)

    w = head_W[0]
    row = lambda a, b: w[a:b].reshape(1, -1)
    w_ue = row(0, 32)
    w_ie = row(32, 64)
    w_pc = row(64, 96)
    w_u_rat = w[96]
    w_pd = row(97, 129)
    w_i_rat = w[129]
    w_g = row(130, 162)
    w_dense = row(162, 170)
    w_gen = row(170, 234)
    w_ug = row(234, 298)

    wrats = jnp.stack([w_u_rat, w_i_rat]).reshape(1, 2)
    densepart = _run_dense_part(dense, w_dense, head_b)
    uh_f, ucnt, urs = _untile_pair(user_hist, user_hist_rat)
    ucomb, uinv, iprojc = _run_proj_u(user_table, user_genome, item_table,
                                      w_ue, w_ug, w_pc, wrats, ucnt, urs)
    ih_f, icnt, irs = _untile_pair(item_hist, item_hist_rat)
    icomb, iinv, uprojd = _run_proj_i(user_table, item_table, movie_genres,
                                      genome, genre_W, w_ie, w_g, w_gen,
                                      w_pd, wrats, icnt, irs)

    return _sc_run(uids, mids, densepart, uh_f, ih_f, ucomb, icomb, uinv,
                   iinv, iprojc, uprojd)
